# Initial kernel scaffold; baseline (speedup 1.0000x reference)
import functools

import jax
import jax.numpy as jnp
from jax import lax
from jax.experimental import pallas as pl
from jax.experimental.pallas import tpu as pltpu
from jax.experimental.pallas import tpu_sc as plsc

N = 10000
E = 8192
D = 128
ED = 16
G = 64
NPAD = 10240
TE = 1024
KG = 8
ER = E // 128


def _sc_gather(x, src2d):
    info = plsc.get_sparse_core_info()
    nc, ns = info.num_cores, info.num_subcores
    nw = nc * ns
    bpw = E // nw
    rpw = bpw // 128
    mesh = plsc.VectorSubcoreMesh(core_axis_name="c", subcore_axis_name="s")

    @functools.partial(
        pl.kernel,
        mesh=mesh,
        out_type=jax.ShapeDtypeStruct((E, D), jnp.float32),
        scratch_types=[
            pltpu.VMEM((rpw, 128), jnp.int32),
            pltpu.VMEM((bpw, D), jnp.float32),
            pltpu.SemaphoreType.DMA,
        ],
    )
    def k(x_hbm, src_hbm, out_hbm, idx_v, rows_v, sem):
        wid = lax.axis_index("s") * nc + lax.axis_index("c")
        pltpu.sync_copy(src_hbm.at[pl.ds(wid * rpw, rpw)], idx_v)
        for j in range(rpw):
            pltpu.async_copy(
                x_hbm.at[idx_v.at[j]], rows_v.at[pl.ds(j * 128, 128)], sem
            ).wait()
        pltpu.sync_copy(rows_v, out_hbm.at[pl.ds(wid * bpw, bpw)])

    return k(x, src2d)


def _sc_scatter(msg, dst2d, z128, z16, ones16):
    info = plsc.get_sparse_core_info()
    nc, ns = info.num_cores, info.num_subcores
    half_rows = ER // nc
    rpw = half_rows // ns
    stripe = NPAD // ns
    mesh = plsc.VectorSubcoreMesh(core_axis_name="c", subcore_axis_name="s")

    @functools.partial(
        pl.kernel,
        mesh=mesh,
        out_type=[
            jax.ShapeDtypeStruct((nc * NPAD, D), jnp.float32),
            jax.ShapeDtypeStruct((nc * NPAD, 16), jnp.float32),
        ],
        scratch_types=[
            pltpu.VMEM((rpw, 128), jnp.int32),
            pltpu.VMEM((rpw * 128, D), jnp.float32),
            pltpu.VMEM((128, 16), jnp.float32),
            pltpu.VMEM_SHARED((NPAD, D), jnp.float32),
            pltpu.VMEM_SHARED((NPAD, 16), jnp.float32),
            pltpu.SemaphoreType.DMA,
        ],
    )
    def k(msg_hbm, dst_hbm, z128_hbm, z16_hbm, ones_hbm, sum_hbm, cnt_hbm,
          idx_v, rows_v, ones_v, sum_sh, cnt_sh, sem):
        cid = lax.axis_index("c")
        sid = lax.axis_index("s")
        pltpu.sync_copy(z128_hbm.at[pl.ds(sid * stripe, stripe)],
                        sum_sh.at[pl.ds(sid * stripe, stripe)])
        pltpu.sync_copy(z16_hbm.at[pl.ds(sid * stripe, stripe)],
                        cnt_sh.at[pl.ds(sid * stripe, stripe)])
        base_rows = cid * half_rows + sid * rpw
        pltpu.sync_copy(dst_hbm.at[pl.ds(base_rows, rpw)], idx_v)
        pltpu.sync_copy(msg_hbm.at[pl.ds(base_rows * 128, rpw * 128)], rows_v)
        pltpu.sync_copy(ones_hbm, ones_v)
        plsc.subcore_barrier()
        for j in range(rpw):
            pltpu.sync_copy(rows_v.at[pl.ds(j * 128, 128)],
                            sum_sh.at[idx_v.at[j]], add=True)
            pltpu.sync_copy(ones_v, cnt_sh.at[idx_v.at[j]], add=True)
        plsc.subcore_barrier()
        pltpu.sync_copy(sum_sh.at[pl.ds(sid * stripe, stripe)],
                        sum_hbm.at[pl.ds(cid * NPAD + sid * stripe, stripe)])
        pltpu.sync_copy(cnt_sh.at[pl.ds(sid * stripe, stripe)],
                        cnt_hbm.at[pl.ds(cid * NPAD + sid * stripe, stripe)])

    return k(msg, dst2d, z128, z16, ones16)


def _mlp_body(ea_ref, w1_ref, b1_ref, w2_ref, b2_ref, w3_ref, b3_ref, h_ref):
    h = jnp.dot(ea_ref[...], w1_ref[...], preferred_element_type=jnp.float32)
    h = jnp.maximum(h + b1_ref[...], 0.0)
    h = jnp.dot(h, w2_ref[...], preferred_element_type=jnp.float32)
    h = jnp.maximum(h + b2_ref[...], 0.0)
    h = jnp.dot(h, w3_ref[...], preferred_element_type=jnp.float32)
    h_ref[...] = jnp.maximum(h + b3_ref[...], 0.0)


def _mlp(edge_attr, w1, b1, w2, b2, w3, b3):
    grid = E // TE
    return pl.pallas_call(
        _mlp_body,
        grid=(grid,),
        in_specs=[
            pl.BlockSpec((TE, ED), lambda i: (i, 0)),
            pl.BlockSpec((ED, 128), lambda i: (0, 0)),
            pl.BlockSpec((1, 128), lambda i: (0, 0)),
            pl.BlockSpec((128, 256), lambda i: (0, 0)),
            pl.BlockSpec((1, 256), lambda i: (0, 0)),
            pl.BlockSpec((256, 128), lambda i: (0, 0)),
            pl.BlockSpec((1, 128), lambda i: (0, 0)),
        ],
        out_specs=pl.BlockSpec((TE, 128), lambda i: (i, 0)),
        out_shape=jax.ShapeDtypeStruct((E, 128), jnp.float32),
    )(edge_attr, w1, b1, w2, b2, w3, b3)


def _einsum_body(h_ref, xs_ref, v_ref, b4_ref, out_ref):
    xs = xs_ref[...]
    h = h_ref[...]
    acc = jnp.dot(xs, b4_ref[...], preferred_element_type=jnp.float32)
    for g in range(128 // KG):
        u = jnp.concatenate(
            [h[:, g * KG + j:g * KG + j + 1] * xs for j in range(KG)], axis=1
        )
        acc = acc + jnp.dot(
            u, v_ref[g * KG * 128:(g + 1) * KG * 128, :],
            preferred_element_type=jnp.float32,
        )
    out_ref[...] = acc


def _einsum(h3, xs, v, b4m):
    grid = E // TE
    return pl.pallas_call(
        _einsum_body,
        grid=(grid,),
        in_specs=[
            pl.BlockSpec((TE, 128), lambda i: (i, 0)),
            pl.BlockSpec((TE, 128), lambda i: (i, 0)),
            pl.BlockSpec((D * D, D), lambda i: (0, 0)),
            pl.BlockSpec((D, D), lambda i: (0, 0)),
        ],
        out_specs=pl.BlockSpec((TE, 128), lambda i: (i, 0)),
        out_shape=jax.ShapeDtypeStruct((E, 128), jnp.float32),
    )(h3, xs, v, b4m)


def _pool_body(sum_ref, cnt_ref, batch_ref, out_ref):
    s = sum_ref[:NPAD, :] + sum_ref[NPAD:, :]
    c = cnt_ref[:NPAD, :1] + cnt_ref[NPAD:, :1]
    node = s * (1.0 / jnp.maximum(c, 1.0))
    oh = (batch_ref[...] ==
          lax.broadcasted_iota(jnp.int32, (NPAD, G), 1)).astype(jnp.float32)
    out_ref[...] = lax.dot_general(
        oh, node, (((0,), (0,)), ((), ())), preferred_element_type=jnp.float32
    )


def _pool(summed, cnt, batchp):
    return pl.pallas_call(
        _pool_body,
        out_shape=jax.ShapeDtypeStruct((G, D), jnp.float32),
    )(summed, cnt, batchp)


def kernel(x, edge_index, edge_attr, batch, W1, b1, W2, b2, W3, b3, W4, b4):
    src2d = edge_index[0].reshape(ER, 128)
    dst2d = edge_index[1].reshape(ER, 128)
    v = W4.reshape(D * D, D)
    b4m = b4.reshape(D, D)
    batchp = jnp.concatenate(
        [batch, jnp.full((NPAD - N,), G, dtype=jnp.int32)]
    ).reshape(NPAD, 1)
    z128 = jnp.zeros((NPAD, D), jnp.float32)
    z16 = jnp.zeros((NPAD, 16), jnp.float32)
    ones16 = jnp.ones((128, 16), jnp.float32)

    h3 = _mlp(edge_attr, W1, b1.reshape(1, 128), W2, b2.reshape(1, 256),
              W3, b3.reshape(1, 128))
    xs = _sc_gather(x, src2d)
    msg = _einsum(h3, xs, v, b4m)
    summed, cnt = _sc_scatter(msg, dst2d, z128, z16, ones16)
    return _pool(summed, cnt, batchp)


# speedup vs baseline: 2.5721x; 2.5721x over previous
import functools

import jax
import jax.numpy as jnp
from jax import lax
from jax.experimental import pallas as pl
from jax.experimental.pallas import tpu as pltpu
from jax.experimental.pallas import tpu_sc as plsc

N = 10000
E = 8192
D = 128
ED = 16
G = 64
NPAD = 10240
TE = 1024
KG = 8
ER = E // 128


def _sc_gather(x, src2d):
    info = plsc.get_sparse_core_info()
    nc, ns = info.num_cores, info.num_subcores
    nw = nc * ns
    bpw = E // nw
    rpw = bpw // 128
    mesh = plsc.VectorSubcoreMesh(core_axis_name="c", subcore_axis_name="s")

    @functools.partial(
        pl.kernel,
        mesh=mesh,
        out_type=jax.ShapeDtypeStruct((E, D), jnp.float32),
        scratch_types=[
            pltpu.VMEM((rpw, 128), jnp.int32),
            pltpu.VMEM((bpw, D), jnp.float32),
            pltpu.SemaphoreType.DMA,
        ],
    )
    def k(x_hbm, src_hbm, out_hbm, idx_v, rows_v, sem):
        wid = lax.axis_index("s") * nc + lax.axis_index("c")
        pltpu.sync_copy(src_hbm.at[pl.ds(wid * rpw, rpw)], idx_v)
        for j in range(rpw):
            pltpu.async_copy(
                x_hbm.at[idx_v.at[j]], rows_v.at[pl.ds(j * 128, 128)], sem
            ).wait()
        pltpu.sync_copy(rows_v, out_hbm.at[pl.ds(wid * bpw, bpw)])

    return k(x, src2d)


def _sc_scatter(msg, dst2d, z128, z16, ones16):
    info = plsc.get_sparse_core_info()
    nc, ns = info.num_cores, info.num_subcores
    half = NPAD // nc
    rpt = ER // ns
    stripe = half // ns
    mesh = plsc.VectorSubcoreMesh(core_axis_name="c", subcore_axis_name="s")

    @functools.partial(
        pl.kernel,
        mesh=mesh,
        out_type=[
            jax.ShapeDtypeStruct((NPAD, D), jnp.float32),
            jax.ShapeDtypeStruct((NPAD, 16), jnp.float32),
        ],
        scratch_types=[
            pltpu.VMEM((1, 128), jnp.int32),
            pltpu.VMEM((128, D), jnp.float32),
            pltpu.VMEM((128, 16), jnp.float32),
            pltpu.VMEM_SHARED((half + 16, D), jnp.float32),
            pltpu.VMEM_SHARED((half + 16, 16), jnp.float32),
            pltpu.SemaphoreType.DMA,
        ],
    )
    def k(msg_hbm, dst_hbm, z128_hbm, z16_hbm, ones_hbm, sum_hbm, cnt_hbm,
          idx_v, rows_v, ones_v, sum_sh, cnt_sh, sem):
        cid = lax.axis_index("c")
        sid = lax.axis_index("s")
        pltpu.sync_copy(z128_hbm.at[pl.ds(sid * stripe, stripe)],
                        sum_sh.at[pl.ds(sid * stripe, stripe)])
        pltpu.sync_copy(z16_hbm.at[pl.ds(sid * stripe, stripe)],
                        cnt_sh.at[pl.ds(sid * stripe, stripe)])
        @pl.when(sid == 0)
        def _():
            pltpu.sync_copy(z128_hbm.at[pl.ds(0, 16)],
                            sum_sh.at[pl.ds(half, 16)])
            pltpu.sync_copy(z16_hbm.at[pl.ds(0, 16)],
                            cnt_sh.at[pl.ds(half, 16)])
        pltpu.sync_copy(ones_hbm, ones_v)
        plsc.subcore_barrier()
        lo = cid * half
        for j in range(rpt):
            row = sid * rpt + j
            pltpu.sync_copy(dst_hbm.at[pl.ds(row, 1)], idx_v)
            for t in range(8):
                v = idx_v[0, pl.ds(t * 16, 16)] - lo
                ok = (v >= 0) & (v < half)
                idx_v[0, pl.ds(t * 16, 16)] = jnp.where(ok, v, half)
            pltpu.sync_copy(msg_hbm.at[pl.ds(row * 128, 128)], rows_v)
            pltpu.sync_copy(rows_v, sum_sh.at[idx_v.at[0]], add=True)
            pltpu.sync_copy(ones_v, cnt_sh.at[idx_v.at[0]], add=True)
        plsc.subcore_barrier()
        pltpu.sync_copy(sum_sh.at[pl.ds(sid * stripe, stripe)],
                        sum_hbm.at[pl.ds(cid * half + sid * stripe, stripe)])
        pltpu.sync_copy(cnt_sh.at[pl.ds(sid * stripe, stripe)],
                        cnt_hbm.at[pl.ds(cid * half + sid * stripe, stripe)])

    return k(msg, dst2d, z128, z16, ones16)


def _mlp_body(ea_ref, w1_ref, b1_ref, w2_ref, b2_ref, w3_ref, b3_ref, h_ref):
    h = jnp.dot(ea_ref[...], w1_ref[...], preferred_element_type=jnp.float32)
    h = jnp.maximum(h + b1_ref[...], 0.0)
    h = jnp.dot(h, w2_ref[...], preferred_element_type=jnp.float32)
    h = jnp.maximum(h + b2_ref[...], 0.0)
    h = jnp.dot(h, w3_ref[...], preferred_element_type=jnp.float32)
    h_ref[...] = jnp.maximum(h + b3_ref[...], 0.0)


def _mlp(edge_attr, w1, b1, w2, b2, w3, b3):
    grid = E // TE
    return pl.pallas_call(
        _mlp_body,
        grid=(grid,),
        in_specs=[
            pl.BlockSpec((TE, ED), lambda i: (i, 0)),
            pl.BlockSpec((ED, 128), lambda i: (0, 0)),
            pl.BlockSpec((1, 128), lambda i: (0, 0)),
            pl.BlockSpec((128, 256), lambda i: (0, 0)),
            pl.BlockSpec((1, 256), lambda i: (0, 0)),
            pl.BlockSpec((256, 128), lambda i: (0, 0)),
            pl.BlockSpec((1, 128), lambda i: (0, 0)),
        ],
        out_specs=pl.BlockSpec((TE, 128), lambda i: (i, 0)),
        out_shape=jax.ShapeDtypeStruct((E, 128), jnp.float32),
    )(edge_attr, w1, b1, w2, b2, w3, b3)


def _einsum_body(h_ref, xs_ref, v_ref, b4_ref, out_ref):
    xs = xs_ref[...]
    h = h_ref[...]
    acc = jnp.dot(xs, b4_ref[...], preferred_element_type=jnp.float32)
    for g in range(128 // KG):
        u = jnp.concatenate(
            [h[:, g * KG + j:g * KG + j + 1] * xs for j in range(KG)], axis=1
        )
        acc = acc + jnp.dot(
            u, v_ref[g * KG * 128:(g + 1) * KG * 128, :],
            preferred_element_type=jnp.float32,
        )
    out_ref[...] = acc


def _einsum(h3, xs, v, b4m):
    grid = E // TE
    return pl.pallas_call(
        _einsum_body,
        grid=(grid,),
        in_specs=[
            pl.BlockSpec((TE, 128), lambda i: (i, 0)),
            pl.BlockSpec((TE, 128), lambda i: (i, 0)),
            pl.BlockSpec((D * D, D), lambda i: (0, 0)),
            pl.BlockSpec((D, D), lambda i: (0, 0)),
        ],
        out_specs=pl.BlockSpec((TE, 128), lambda i: (i, 0)),
        out_shape=jax.ShapeDtypeStruct((E, 128), jnp.float32),
    )(h3, xs, v, b4m)


def _pool_body(sum_ref, cnt_ref, batch_ref, out_ref):
    s = sum_ref[...]
    c = cnt_ref[:, :1]
    node = s * (1.0 / jnp.maximum(c, 1.0))
    oh = (batch_ref[...] ==
          lax.broadcasted_iota(jnp.int32, (NPAD, G), 1)).astype(jnp.float32)
    out_ref[...] = lax.dot_general(
        oh, node, (((0,), (0,)), ((), ())), preferred_element_type=jnp.float32
    )


def _pool(summed, cnt, batchp):
    return pl.pallas_call(
        _pool_body,
        out_shape=jax.ShapeDtypeStruct((G, D), jnp.float32),
    )(summed, cnt, batchp)


def kernel(x, edge_index, edge_attr, batch, W1, b1, W2, b2, W3, b3, W4, b4):
    src2d = edge_index[0].reshape(ER, 128)
    dst2d = edge_index[1].reshape(ER, 128)
    v = W4.reshape(D * D, D)
    b4m = b4.reshape(D, D)
    batchp = jnp.concatenate(
        [batch, jnp.full((NPAD - N,), G, dtype=jnp.int32)]
    ).reshape(NPAD, 1)
    z128 = jnp.zeros((NPAD, D), jnp.float32)
    z16 = jnp.zeros((NPAD, 16), jnp.float32)
    ones16 = jnp.ones((128, 16), jnp.float32)

    h3 = _mlp(edge_attr, W1, b1.reshape(1, 128), W2, b2.reshape(1, 256),
              W3, b3.reshape(1, 128))
    xs = _sc_gather(x, src2d)
    msg = _einsum(h3, xs, v, b4m)
    summed, cnt = _sc_scatter(msg, dst2d, z128, z16, ones16)
    return _pool(summed, cnt, batchp)
